# baseline (speedup 1.0000x reference)
import functools

import jax
import jax.numpy as jnp
from jax import lax
from jax.experimental import pallas as pl
from jax.experimental.pallas import tpu as pltpu
from jax.experimental.pallas import tpu_sc as plsc

PAD = 128
ROW_BLK = 20480
SC_CORES = 2
SC_SUBCORES = 16
LANES = 16


def _bf16_bits(u):
    return (u + jnp.uint32(0x7FFF) + ((u >> 16) & jnp.uint32(1))) >> 16


def _table_mlp_body(emb_ref, w1_ref, b1_ref, w2_ref, b2_ref, w3_ref, b3_ref,
                    out_ref):
    h = jnp.tanh(
        jnp.dot(emb_ref[...], w1_ref[...], preferred_element_type=jnp.float32)
        + b1_ref[...])
    h = jnp.tanh(
        jnp.dot(h, w2_ref[...], preferred_element_type=jnp.float32)
        + b2_ref[...])
    st = lax.dot_general(w3_ref[...], h, (((0,), (1,)), ((), ())),
                         preferred_element_type=jnp.float32) + b3_ref[...]
    u = lax.bitcast_convert_type(st, jnp.uint32)
    r0 = _bf16_bits(u[0:1, :])
    r1 = _bf16_bits(u[1:2, :])
    packed = r0 | (r1 << 16)
    out_ref[...] = lax.bitcast_convert_type(packed, jnp.int32).reshape(
        packed.shape[1])


def _score_table(emb, W1, b1, W2, b2, W3, b3):
    V, E = emb.shape
    H = W1.shape[1]
    O = W3.shape[1]
    grid = (pl.cdiv(V, ROW_BLK),)
    return pl.pallas_call(
        _table_mlp_body,
        grid=grid,
        in_specs=[
            pl.BlockSpec((ROW_BLK, E), lambda i: (i, 0)),
            pl.BlockSpec((E, H), lambda i: (0, 0)),
            pl.BlockSpec((1, H), lambda i: (0, 0)),
            pl.BlockSpec((H, H), lambda i: (0, 0)),
            pl.BlockSpec((1, H), lambda i: (0, 0)),
            pl.BlockSpec((H, O), lambda i: (0, 0)),
            pl.BlockSpec((O, 1), lambda i: (0, 0)),
        ],
        out_specs=pl.BlockSpec((ROW_BLK,), lambda i: (i,)),
        out_shape=jax.ShapeDtypeStruct((V,), jnp.int32),
    )(emb, W1, b1, W2, b2, W3, b3)


def _sc_gather(table, idx, batch):
    n = idx.shape[0]
    v = table.shape[0]
    nw = SC_CORES * SC_SUBCORES
    per_worker = n // nw
    mesh = plsc.VectorSubcoreMesh(core_axis_name="core",
                                  subcore_axis_name="subcore")
    params = pltpu.CompilerParams(needs_layout_passes=False)

    b_per_w = batch // nw
    f_count = per_worker // b_per_w

    @functools.partial(
        pl.kernel,
        out_type=jax.ShapeDtypeStruct((n,), jnp.int32),
        mesh=mesh,
        compiler_params=params,
        scratch_types=[
            pltpu.VMEM((v,), jnp.int32),
            pltpu.VMEM((per_worker,), jnp.int32),
            pltpu.VMEM((per_worker,), jnp.int32),
            pltpu.SemaphoreType.DMA,
            pltpu.SemaphoreType.DMA,
        ])
    def gather_kernel(tab_hbm, i_hbm, o_hbm, tab_v, idx_v, out_v, tsem, sem):
        tab_cp = pltpu.make_async_copy(tab_hbm, tab_v, tsem)
        tab_cp.start()
        wid = lax.axis_index("core") * SC_SUBCORES + lax.axis_index("subcore")
        base0 = wid * per_worker
        pltpu.sync_copy(i_hbm.at[pl.ds(base0, per_worker)], idx_v)
        tab_cp.wait()
        lane_iota = lax.iota(jnp.int32, LANES)

        @pl.loop(0, per_worker, step=4 * LANES)
        def _(t):
            for j in range(4):
                o = t + j * LANES
                g16 = o + lane_iota
                idx16 = idx_v[pl.ds(o, LANES)]
                vals16 = plsc.load_gather(tab_v, [idx16])
                b16 = g16 // f_count
                f16 = g16 - b16 * f_count
                pos16 = f16 * b_per_w + b16
                plsc.store_scatter(out_v, [pos16], vals16)

        cps = []
        for f in range(f_count):
            cps.append(pltpu.make_async_copy(
                out_v.at[pl.ds(f * b_per_w, b_per_w)],
                o_hbm.at[pl.ds(f * batch + wid * b_per_w, b_per_w)],
                sem))
        for cp in cps:
            cp.start()
        for cp in cps:
            cp.wait()

    return gather_kernel(table, idx)


def kernel(x, emb, W1, b1, W2, b2, W3, b3):
    B_, F_ = x.shape
    H, O = W3.shape
    table = _score_table(emb, W1, b1.reshape(1, -1), W2, b2.reshape(1, -1),
                         W3, b3.reshape(-1, 1))
    idx = x.reshape(-1).astype(jnp.int32)
    packed = _sc_gather(table, idx, B_)
    pair = lax.bitcast_convert_type(packed, jnp.bfloat16)
    return pair.astype(jnp.float32).reshape(F_, B_, O).transpose(1, 0, 2)


# speedup vs baseline: 1.0005x; 1.0005x over previous
import functools

import jax
import jax.numpy as jnp
from jax import lax
from jax.experimental import pallas as pl
from jax.experimental.pallas import tpu as pltpu
from jax.experimental.pallas import tpu_sc as plsc

ROW_BLK = 20480
SC_CORES = 2
SC_SUBCORES = 16
LANES = 16


def _bf16_bits(u):
    return (u + jnp.uint32(0x7FFF) + ((u >> 16) & jnp.uint32(1))) >> 16


def _table_mlp_body(emb_ref, w1_ref, b1_ref, w2_ref, b2_ref, w3_ref, b3_ref,
                    out_ref):
    h = jnp.tanh(
        jnp.dot(emb_ref[...], w1_ref[...], preferred_element_type=jnp.float32)
        + b1_ref[...])
    h = jnp.tanh(
        jnp.dot(h, w2_ref[...], preferred_element_type=jnp.float32)
        + b2_ref[...])
    st = lax.dot_general(w3_ref[...], h, (((0,), (1,)), ((), ())),
                         preferred_element_type=jnp.float32) + b3_ref[...]
    u = lax.bitcast_convert_type(st, jnp.uint32)
    r0 = _bf16_bits(u[0:1, :])
    r1 = _bf16_bits(u[1:2, :])
    packed = r0 | (r1 << 16)
    out_ref[...] = lax.bitcast_convert_type(packed, jnp.int32).reshape(
        packed.shape[1])


def _score_table(emb, W1, b1, W2, b2, W3, b3):
    V, E = emb.shape
    H = W1.shape[1]
    O = W3.shape[1]
    grid = (pl.cdiv(V, ROW_BLK),)
    return pl.pallas_call(
        _table_mlp_body,
        grid=grid,
        in_specs=[
            pl.BlockSpec((ROW_BLK, E), lambda i: (i, 0)),
            pl.BlockSpec((E, H), lambda i: (0, 0)),
            pl.BlockSpec((1, H), lambda i: (0, 0)),
            pl.BlockSpec((H, H), lambda i: (0, 0)),
            pl.BlockSpec((1, H), lambda i: (0, 0)),
            pl.BlockSpec((H, O), lambda i: (0, 0)),
            pl.BlockSpec((O, 1), lambda i: (0, 0)),
        ],
        out_specs=pl.BlockSpec((ROW_BLK,), lambda i: (i,)),
        out_shape=jax.ShapeDtypeStruct((V,), jnp.int32),
    )(emb, W1, b1, W2, b2, W3, b3)


def _sc_gather(table, idx, batch):
    n = idx.shape[0]
    v = table.shape[0]
    nw = SC_CORES * SC_SUBCORES
    per_worker = n // nw
    mesh = plsc.VectorSubcoreMesh(core_axis_name="core",
                                  subcore_axis_name="subcore")
    params = pltpu.CompilerParams(needs_layout_passes=False)

    b_per_w = batch // nw
    f_count = per_worker // b_per_w

    @functools.partial(
        pl.kernel,
        out_type=jax.ShapeDtypeStruct((n,), jnp.int32),
        mesh=mesh,
        compiler_params=params,
        scratch_types=[
            pltpu.VMEM((v,), jnp.int32),
            pltpu.VMEM((per_worker,), jnp.int32),
            pltpu.VMEM((per_worker,), jnp.int32),
            pltpu.SemaphoreType.DMA,
            pltpu.SemaphoreType.DMA,
        ])
    def gather_kernel(tab_hbm, i_hbm, o_hbm, tab_v, idx_v, out_v, tsem, sem):
        tab_cp = pltpu.make_async_copy(tab_hbm, tab_v, tsem)
        tab_cp.start()
        wid = lax.axis_index("core") * SC_SUBCORES + lax.axis_index("subcore")
        base0 = wid * per_worker
        pltpu.sync_copy(i_hbm.at[pl.ds(base0, per_worker)], idx_v)
        tab_cp.wait()
        lane_iota = lax.iota(jnp.int32, LANES)

        @pl.loop(0, per_worker, step=4 * LANES)
        def _(t):
            for j in range(4):
                o = t + j * LANES
                g16 = o + lane_iota
                idx16 = idx_v[pl.ds(o, LANES)]
                vals16 = plsc.load_gather(tab_v, [idx16])
                b16 = g16 // f_count
                f16 = g16 - b16 * f_count
                pos16 = f16 * b_per_w + b16
                plsc.store_scatter(out_v, [pos16], vals16)

        cps = []
        for f in range(f_count):
            cps.append(pltpu.make_async_copy(
                out_v.at[pl.ds(f * b_per_w, b_per_w)],
                o_hbm.at[pl.ds(f * batch + wid * b_per_w, b_per_w)],
                sem))
        for cp in cps:
            cp.start()
        for cp in cps:
            cp.wait()

    return gather_kernel(table, idx)


def kernel(x, emb, W1, b1, W2, b2, W3, b3):
    B_, F_ = x.shape
    H, O = W3.shape
    table = _score_table(emb, W1, b1.reshape(1, -1), W2, b2.reshape(1, -1),
                         W3, b3.reshape(-1, 1))
    idx = x.reshape(-1).astype(jnp.int32)
    packed = _sc_gather(table, idx, B_)
    pair = lax.bitcast_convert_type(packed, jnp.bfloat16)
    return pair.astype(jnp.float32).reshape(F_, B_, O).transpose(1, 0, 2)
